# Initial kernel scaffold; baseline (speedup 1.0000x reference)
import jax
import jax.numpy as jnp
from jax.experimental import pallas as pl


def kernel(mh_output, W_route, b_route, W_noise, b_noise):
    raise NotImplementedError("write your pallas kernel here")


# speedup vs baseline: 1.4989x; 1.4989x over previous
import functools

import jax
import jax.numpy as jnp
from jax import lax
from jax.experimental import pallas as pl
from jax.experimental.pallas import tpu as pltpu
from jax.experimental.pallas import tpu_sc as plsc

_NEG_INF = float("-inf")


def _noisy_kernel(x_ref, w_ref, b_ref, ns_ref, out_ref, *, num_experts):
    z = jnp.dot(x_ref[...], w_ref[...], preferred_element_type=jnp.float32)
    z = z + b_ref[...]
    logits = z[:, :num_experts]
    noise_logits = z[:, num_experts:]
    out_ref[...] = logits + ns_ref[...] * jax.nn.softplus(noise_logits)


def _noisy_logits(x, w_c, b_c, noise_sample, *, block_tokens):
    n, d = x.shape
    e2 = w_c.shape[1]
    e = e2 // 2
    grid = n // block_tokens
    return pl.pallas_call(
        functools.partial(_noisy_kernel, num_experts=e),
        grid=(grid,),
        in_specs=[
            pl.BlockSpec((block_tokens, d), lambda i: (i, 0)),
            pl.BlockSpec((d, e2), lambda i: (0, 0)),
            pl.BlockSpec((1, e2), lambda i: (0, 0)),
            pl.BlockSpec((block_tokens, e), lambda i: (i, 0)),
        ],
        out_specs=pl.BlockSpec((block_tokens, e), lambda i: (i, 0)),
        out_shape=jax.ShapeDtypeStruct((n, e), jnp.float32),
        compiler_params=pltpu.CompilerParams(
            dimension_semantics=("arbitrary",),
        ),
    )(x, w_c, b_c, noise_sample)


def _make_route(n_tokens, num_experts):
    info = plsc.get_sparse_core_info()
    nc, ns, nl = info.num_cores, info.num_subcores, info.num_lanes
    nw = nc * ns
    assert n_tokens % (nw * nl) == 0
    tpw = n_tokens // nw
    groups = tpw // nl

    mesh = plsc.VectorSubcoreMesh(core_axis_name="c", subcore_axis_name="s")

    @functools.partial(
        pl.kernel,
        mesh=mesh,
        out_type=[
            jax.ShapeDtypeStruct((n_tokens * num_experts,), jnp.float32),
            jax.ShapeDtypeStruct((n_tokens * 2,), jnp.int32),
        ],
        scratch_types=[
            pltpu.VMEM((tpw * num_experts,), jnp.float32),
            pltpu.VMEM((tpw * num_experts,), jnp.float32),
            pltpu.VMEM((tpw * 2,), jnp.int32),
        ],
        compiler_params=pltpu.CompilerParams(needs_layout_passes=False),
    )
    def route(noisy_hbm, probs_hbm, idx_hbm, nv, pv, iv):
        wid = lax.axis_index("s") * nc + lax.axis_index("c")
        base = wid * tpw
        pltpu.sync_copy(noisy_hbm.at[pl.ds(base * num_experts, tpw * num_experts)], nv)

        lanes = lax.iota(jnp.int32, nl)
        neg_inf = jnp.full((nl,), _NEG_INF, jnp.float32)

        def body(g, carry):
            rows = g * nl + lanes
            rows_e = rows * num_experts
            v = [
                plsc.load_gather(nv, [rows_e + e])
                for e in range(num_experts)
            ]
            best1 = v[0]
            idx1 = jnp.zeros((nl,), jnp.int32)
            for e in range(1, num_experts):
                c = v[e] > best1
                best1 = jnp.where(c, v[e], best1)
                idx1 = jnp.where(c, jnp.full((nl,), e, jnp.int32), idx1)
            best2 = neg_inf
            idx2 = jnp.zeros((nl,), jnp.int32)
            for e in range(num_experts):
                ee = jnp.full((nl,), e, jnp.int32)
                cand = jnp.where(idx1 == ee, neg_inf, v[e])
                c = cand > best2
                best2 = jnp.where(c, cand, best2)
                idx2 = jnp.where(c, ee, idx2)
            e2 = jnp.exp(best2 - best1)
            denom = 1.0 + e2
            p1 = 1.0 / denom
            p2 = e2 / denom
            zero = jnp.zeros((nl,), jnp.float32)
            for e in range(num_experts):
                ee = jnp.full((nl,), e, jnp.int32)
                pe = jnp.where(idx1 == ee, p1, jnp.where(idx2 == ee, p2, zero))
                plsc.store_scatter(pv, [rows_e + e], pe)
            rows2 = rows * 2
            plsc.store_scatter(iv, [rows2], idx1)
            plsc.store_scatter(iv, [rows2 + 1], idx2)
            return carry

        lax.fori_loop(0, groups, body, 0)
        pltpu.sync_copy(pv, probs_hbm.at[pl.ds(base * num_experts, tpw * num_experts)])
        pltpu.sync_copy(iv, idx_hbm.at[pl.ds(base * 2, tpw * 2)])

    return route


def kernel(mh_output, W_route, b_route, W_noise, b_noise):
    b, t, d = mh_output.shape
    e = W_route.shape[0]
    n = b * t

    x = mh_output.reshape(n, d)
    w_c = jnp.concatenate([W_route, W_noise], axis=0).T
    b_c = jnp.concatenate([b_route, b_noise], axis=0).reshape(1, 2 * e)
    noise_sample = jax.random.normal(
        jax.random.key(42), (b, t, e), dtype=jnp.float32
    ).reshape(n, e)

    noisy = _noisy_logits(x, w_c, b_c, noise_sample, block_tokens=2048)
    probs, idx = _make_route(n, e)(noisy.reshape(n * e))
    return probs.reshape(b, t, e), idx.reshape(b, t, 2)
